# Initial kernel scaffold; baseline (speedup 1.0000x reference)
import jax
import jax.numpy as jnp
from jax.experimental import pallas as pl


def kernel(features, edge_index, params):
    raise NotImplementedError("write your pallas kernel here")


# speedup vs baseline: 11.6653x; 11.6653x over previous
import jax
import jax.numpy as jnp
from jax.experimental import pallas as pl

N = 300
E = 9600
HID = 64
AH = 32
CH = 1200
NCH = E // CH
F32 = jnp.float32
EPS = 1e-5


def _fwd_body(edge, feat,
              W01, b01, ga1, be1, W11, b11,
              W02, b02, ga2, be2, W12, b12,
              WaS, WaA, WaB, ba0T, Wa1r,
              Wc0, bc0, Wc1, bc1,
              pi_ref, val_ref):
    def onehot_t(row):
        ids = jax.lax.broadcasted_iota(jnp.int32, (N, CH), 0)
        return (row == ids).astype(jnp.bfloat16)

    adj = jnp.zeros((N, N), F32)
    for c in range(NCH):
        s = edge[0:1, c * CH:(c + 1) * CH]
        d = edge[1:2, c * CH:(c + 1) * CH]
        adj = adj + jax.lax.dot_general(
            onehot_t(d), onehot_t(s),
            (((1,), (1,)), ((), ())), preferred_element_type=F32)

    def gin(x, W0, b0, ga, be, W1, b1):
        xa = x + jnp.dot(adj, x, preferred_element_type=F32)
        h = jnp.dot(xa, W0[...], preferred_element_type=F32) + b0[...]
        mu = jnp.mean(h, axis=0, keepdims=True)
        var = jnp.mean((h - mu) ** 2, axis=0, keepdims=True)
        h = ga[...] * (h - mu) / jnp.sqrt(var + EPS) + be[...]
        h = jnp.maximum(h, 0.0)
        return jnp.dot(h, W1[...], preferred_element_type=F32) + b1[...]

    x1 = gin(feat[...], W01, b01, ga1, be1, W11, b11)
    x2 = gin(x1, W02, b02, ga2, be2, W12, b12)

    g = jnp.mean(x2, axis=0, keepdims=True)

    hc = jnp.maximum(jnp.dot(g, Wc0[...], preferred_element_type=F32)
                     + bc0[...], 0.0)
    val_ref[...] = jnp.dot(hc, Wc1[...], preferred_element_type=F32) + bc1[...]

    AT = jax.lax.dot_general(WaA[...], x2, (((0,), (1,)), ((), ())),
                             preferred_element_type=F32)
    gAT = jax.lax.dot_general(WaS[...], g, (((0,), (1,)), ((), ())),
                              preferred_element_type=F32)
    AT2 = AT + gAT + ba0T[...]
    B = jnp.dot(x2, WaB[...], preferred_element_type=F32)

    L = jnp.zeros((N, N), F32)
    for k in range(AH):
        zk = AT2[k:k + 1, :] + B[:, k:k + 1]
        L = L + jnp.maximum(zk, 0.0) * Wa1r[0:1, k:k + 1]
    m = jnp.max(L, keepdims=True)
    ex = jnp.exp(L - m)
    pi_ref[...] = ex / jnp.sum(ex, keepdims=True)


_OUT_SHAPE = (jax.ShapeDtypeStruct((N, N), F32),
              jax.ShapeDtypeStruct((1, 1), F32))


def _prep_args(features, edge_index, params):
    gp = params['gin']
    ap = params['actor']
    cp = params['critic']
    featp = jnp.zeros((N, 8), F32).at[:, :2].set(features)
    W01 = jnp.zeros((8, HID), F32).at[:2, :].set(gp[0]['W0'])
    r = lambda a: a.reshape(1, -1)
    Wa0 = ap['W0']
    return [
        edge_index, featp,
        W01, r(gp[0]['b0']), r(gp[0]['gamma']), r(gp[0]['beta']),
        gp[0]['W1'], r(gp[0]['b1']),
        gp[1]['W0'], r(gp[1]['b0']), r(gp[1]['gamma']), r(gp[1]['beta']),
        gp[1]['W1'], r(gp[1]['b1']),
        Wa0[0:HID, :], Wa0[HID:2 * HID, :], Wa0[2 * HID:3 * HID, :],
        ap['b0'].reshape(AH, 1), ap['W1'].reshape(1, AH),
        cp['W0'], r(cp['b0']), cp['W1'], cp['b1'].reshape(1, 1),
    ]


def kernel(features, edge_index, params):
    args = _prep_args(features, edge_index, params)
    pi300, val = pl.pallas_call(_fwd_body, out_shape=_OUT_SHAPE)(*args)
    return (pi300.reshape(N * N, 1), val)


# speedup vs baseline: 13.5771x; 1.1639x over previous
import jax
import jax.numpy as jnp
from jax.experimental import pallas as pl

N = 300
E = 9600
HID = 64
AH = 32
CH = 1200
NCH = E // CH
F32 = jnp.float32
EPS = 1e-5


def _fwd_body(edge, feat,
              W01, b01, ga1, be1, W11, b11,
              W02, b02, ga2, be2, W12, b12,
              Wa0, ba0, Wa1,
              Wc0, bc0, Wc1, bc1,
              pi_ref, val_ref):
    def onehot_t(row):
        ids = jax.lax.broadcasted_iota(jnp.int32, (N, CH), 0)
        return (row == ids).astype(jnp.bfloat16)

    adj = jnp.zeros((N, N), F32)
    for c in range(NCH):
        s = edge[0:1, c * CH:(c + 1) * CH]
        d = edge[1:2, c * CH:(c + 1) * CH]
        adj = adj + jax.lax.dot_general(
            onehot_t(d), onehot_t(s),
            (((1,), (1,)), ((), ())), preferred_element_type=F32)

    def gin(x, W0, b0, ga, be, W1, b1):
        xa = x + jnp.dot(adj, x, preferred_element_type=F32)
        h = jnp.dot(xa, W0[...], preferred_element_type=F32) + b0[...]
        mu = jnp.mean(h, axis=0, keepdims=True)
        var = jnp.mean((h - mu) ** 2, axis=0, keepdims=True)
        h = ga[...] * (h - mu) / jnp.sqrt(var + EPS) + be[...]
        h = jnp.maximum(h, 0.0)
        return jnp.dot(h, W1[...], preferred_element_type=F32) + b1[...]

    x1 = gin(feat[...], W01, b01, ga1, be1, W11, b11)
    x2 = gin(x1, W02, b02, ga2, be2, W12, b12)

    g = jnp.mean(x2, axis=0, keepdims=True)

    hc = jnp.maximum(jnp.dot(g, Wc0[...], preferred_element_type=F32)
                     + bc0[...], 0.0)
    val_ref[...] = jnp.dot(hc, Wc1[...], preferred_element_type=F32) + bc1[...]

    AT = jax.lax.dot_general(Wa0[HID:2 * HID, :], x2, (((0,), (1,)), ((), ())),
                             preferred_element_type=F32)
    B = jnp.dot(x2, Wa0[2 * HID:, :], preferred_element_type=F32)
    gA = jnp.dot(g, Wa0[:HID, :], preferred_element_type=F32) \
        + ba0[...][None, :]

    L = jnp.zeros((N, N), F32)
    for k in range(AH):
        zk = AT[k:k + 1, :] + B[:, k:k + 1] + gA[0:1, k:k + 1]
        L = L + jnp.maximum(zk, 0.0) * Wa1[k:k + 1, 0:1]
    m = jnp.max(L, keepdims=True)
    ex = jnp.exp(L - m)
    pi_ref[...] = ex / jnp.sum(ex, keepdims=True)


_OUT_SHAPE = (jax.ShapeDtypeStruct((N, N), F32),
              jax.ShapeDtypeStruct((1, 1), F32))


def _flat_args(features, edge_index, params):
    gp = params['gin']
    ap = params['actor']
    cp = params['critic']
    return [
        edge_index, features,
        gp[0]['W0'], gp[0]['b0'], gp[0]['gamma'], gp[0]['beta'],
        gp[0]['W1'], gp[0]['b1'],
        gp[1]['W0'], gp[1]['b0'], gp[1]['gamma'], gp[1]['beta'],
        gp[1]['W1'], gp[1]['b1'],
        ap['W0'], ap['b0'], ap['W1'],
        cp['W0'], cp['b0'], cp['W1'], cp['b1'].reshape(1, 1),
    ]


def kernel(features, edge_index, params):
    args = _flat_args(features, edge_index, params)
    pi300, val = pl.pallas_call(_fwd_body, out_shape=_OUT_SHAPE)(*args)
    return (pi300.reshape(N * N, 1), val)
